# baseline (speedup 1.0000x reference)
import functools

import jax
import jax.numpy as jnp
from jax import lax
from jax.experimental import pallas as pl
from jax.experimental.pallas import tpu as pltpu
from jax.experimental.pallas import tpu_sc as plsc

B = 16384
GV = 1000000
LV = 100000
DIM = 32

GTAIL = (GV // 512) * 512
LTAIL0 = (LV // 512) * 512
LTAIL = (LV // 128) * 128
GW = GV // 512
LW = LV // 512


def _transpose_window(slabs, stage):
    rows8 = lax.iota(jnp.int32, 16) & 7
    vsel = lax.iota(jnp.int32, 16) >> 3

    def pair_body(k, carry):
        for u in range(4):
            vloc = (4 * k + u) * 2
            gcols = vloc + vsel
            srow = jnp.broadcast_to(vloc >> 2, (16,))
            scol0 = (vloc & 3) * 32 + rows8 + vsel * 32
            for s in range(4):
                v = plsc.load_gather(slabs[s], [rows8, gcols])
                plsc.store_scatter(stage, [srow, scol0 + 8 * s], v)
        return carry

    lax.fori_loop(0, 64, pair_body, 0)


@functools.lru_cache(maxsize=1)
def _build():
    info = plsc.get_sparse_core_info()
    NC, NS = info.num_cores, info.num_subcores
    NW = NC * NS
    mesh = plsc.VectorSubcoreMesh(core_axis_name="c", subcore_axis_name="s")
    cp = pltpu.CompilerParams(needs_layout_passes=False)

    @functools.partial(
        pl.kernel,
        mesh=mesh,
        compiler_params=cp,
        out_type=(
            jax.ShapeDtypeStruct((GV // 4, 128), jnp.float32),
            jax.ShapeDtypeStruct((GV // 4, 128), jnp.float32),
            jax.ShapeDtypeStruct((LV // 4, 128), jnp.float32),
            jax.ShapeDtypeStruct((LV // 4, 128), jnp.float32),
        ),
        scratch_types=[pltpu.VMEM((8, 512), jnp.float32) for _ in range(8)]
        + [
            pltpu.VMEM((128, 128), jnp.float32),
            pltpu.VMEM((128, 128), jnp.float32),
            pltpu.SemaphoreType.DMA,
            pltpu.SemaphoreType.DMA,
        ],
    )
    def retile(WuT, WiT, WaT, WbT, tu, ti, ta, tb, Du, Di, Da, Db,
               s00, s01, s02, s03, s10, s11, s12, s13, st0, st1,
               semA, semB):
        wid = lax.axis_index("s") * NC + lax.axis_index("c")
        slabs = ((s00, s01, s02, s03), (s10, s11, s12, s13))
        stages = (st0, st1)
        sems = (semA, semB)

        def stream_table(WT, D, nwin, per_w):
            w0 = (wid * nwin) // NW
            clamp = nwin - 1

            def issue(j, p):
                win = jnp.minimum(w0 + j, clamp)
                for s in range(4):
                    pltpu.async_copy(
                        WT.at[pl.ds(8 * s, 8), pl.ds(512 * win, 512)],
                        slabs[p][s], sems[p])

            def drain(p):
                for s in range(4):
                    pltpu.make_async_copy(
                        WT.at[pl.ds(0, 8), pl.ds(0, 512)],
                        slabs[p][s], sems[p]).wait()

            issue(0, 0)
            issue(1, 1)

            def pair(jp, carry):
                for p in range(2):
                    j = 2 * jp + p
                    win = jnp.minimum(w0 + j, clamp)
                    drain(p)
                    _transpose_window(slabs[p], stages[p])
                    issue(j + 2, p)
                    pltpu.sync_copy(stages[p],
                                    D.at[pl.ds(128 * win, 128)])
                return carry

            lax.fori_loop(0, (per_w + 1) // 2, pair, 0)
            drain(0)
            drain(1)

        stream_table(WuT, Du, GW, 62)
        stream_table(WiT, Di, GW, 62)
        stream_table(WaT, Da, LW, 7)
        stream_table(WbT, Db, LW, 7)

        def odd_tile(WT, D):
            for s in range(4):
                pltpu.sync_copy(
                    WT.at[pl.ds(8 * s, 8), pl.ds(LTAIL0, 128)],
                    slabs[0][s].at[:, pl.ds(0, 128)])
            rows8 = lax.iota(jnp.int32, 16) & 7
            vsel = lax.iota(jnp.int32, 16) >> 3

            def pb(k, carry):
                for u in range(4):
                    vloc = (4 * k + u) * 2
                    srow = jnp.broadcast_to(vloc >> 2, (16,))
                    base = (vloc & 3) * 32 + rows8 + vsel * 32
                    for s in range(4):
                        v = plsc.load_gather(slabs[0][s], [rows8, vloc + vsel])
                        plsc.store_scatter(st0, [srow, base + 8 * s], v)
                return carry

            lax.fori_loop(0, 16, pb, 0)
            pltpu.sync_copy(st0.at[pl.ds(0, 32)],
                            D.at[pl.ds(LTAIL0 // 4, 32)])

        @pl.when(wid == 1)
        def _():
            odd_tile(WaT, Da)

        @pl.when(wid == 2)
        def _():
            odd_tile(WbT, Db)

        @pl.when(wid == 3)
        def _():
            pltpu.sync_copy(tu, st0.at[pl.ds(0, 16)])
            pltpu.sync_copy(st0.at[pl.ds(0, 16)],
                            Du.at[pl.ds(GTAIL // 4, 16)])

        @pl.when(wid == 4)
        def _():
            pltpu.sync_copy(ti, st0.at[pl.ds(0, 16)])
            pltpu.sync_copy(st0.at[pl.ds(0, 16)],
                            Di.at[pl.ds(GTAIL // 4, 16)])

        @pl.when(wid == 5)
        def _():
            pltpu.sync_copy(ta, st0.at[pl.ds(0, 8)])
            pltpu.sync_copy(st0.at[pl.ds(0, 8)],
                            Da.at[pl.ds(LTAIL // 4, 8)])

        @pl.when(wid == 6)
        def _():
            pltpu.sync_copy(tb, st0.at[pl.ds(0, 8)])
            pltpu.sync_copy(st0.at[pl.ds(0, 8)],
                            Db.at[pl.ds(LTAIL // 4, 8)])

    NB = 128

    @functools.partial(
        pl.kernel,
        mesh=mesh,
        compiler_params=cp,
        out_type=(
            jax.ShapeDtypeStruct((2 * DIM, B), jnp.float32),
            jax.ShapeDtypeStruct((2 * DIM, B), jnp.float32),
        ),
        scratch_types=[pltpu.VMEM((512,), jnp.int32) for _ in range(4)]
        + [pltpu.VMEM((NB,), jnp.int32) for _ in range(4)]
        + [pltpu.VMEM((NB, 128), jnp.float32) for _ in range(4)]
        + [
            pltpu.VMEM((2 * DIM, NB), jnp.float32),
            pltpu.VMEM((2 * DIM, NB), jnp.float32),
            pltpu.SemaphoreType.DMA,
        ],
    )
    def gather(Du, Di, Da, Db, uid, iid, ca, cb, gT, lT,
               xu, xi, xa, xb, qu, qi, qa, qb,
               ru, ri, ra, rb, ag, al, sem):
        wid = lax.axis_index("s") * NC + lax.axis_index("c")
        base = wid * 512
        pltpu.sync_copy(uid.at[pl.ds(base, 512)], xu)
        pltpu.sync_copy(iid.at[pl.ds(base, 512)], xi)
        pltpu.sync_copy(ca.at[pl.ds(base, 512)], xa)
        pltpu.sync_copy(cb.at[pl.ds(base, 512)], xb)
        lanes = lax.iota(jnp.int32, 16)

        for c in range(512 // NB):
            for x, q in ((xu, qu), (xi, qi), (xa, qa), (xb, qb)):
                for j in range(NB // 16):
                    q[pl.ds(16 * j, 16)] = x[pl.ds(NB * c + 16 * j, 16)] >> 2
            cps = [pltpu.async_copy(D.at[q], r, sem)
                   for D, q, r in ((Du, qu, ru), (Di, qi, ri),
                                   (Da, qa, ra), (Db, qb, rb))]
            for h in cps:
                h.wait()
            for x, q in ((xu, qu), (xi, qi), (xa, qa), (xb, qb)):
                for j in range(NB // 16):
                    q[pl.ds(16 * j, 16)] = (
                        x[pl.ds(NB * c + 16 * j, 16)] & 3) * 32

            def f_body(f, carry):
                for j in range(NB // 16):
                    rows = lanes + 16 * j
                    ag[f, pl.ds(16 * j, 16)] = plsc.load_gather(
                        ru, [rows, qu[pl.ds(16 * j, 16)] + f])
                    ag[DIM + f, pl.ds(16 * j, 16)] = plsc.load_gather(
                        ri, [rows, qi[pl.ds(16 * j, 16)] + f])
                    al[f, pl.ds(16 * j, 16)] = plsc.load_gather(
                        ra, [rows, qa[pl.ds(16 * j, 16)] + f])
                    al[DIM + f, pl.ds(16 * j, 16)] = plsc.load_gather(
                        rb, [rows, qb[pl.ds(16 * j, 16)] + f])
                return carry

            lax.fori_loop(0, DIM, f_body, 0)
            pltpu.sync_copy(ag, gT.at[pl.ds(0, 2 * DIM),
                                      pl.ds(base + NB * c, NB)])
            pltpu.sync_copy(al, lT.at[pl.ds(0, 2 * DIM),
                                      pl.ds(base + NB * c, NB)])

    return retile, gather


def kernel(W_user, W_item, W_cat_a, W_cat_b, user_id, item_id, cat_a, cat_b):
    retile, gather = _build()
    tu = W_user[GTAIL:].reshape(16, 128)
    ti = W_item[GTAIL:].reshape(16, 128)
    ta = W_cat_a[LTAIL:].reshape(8, 128)
    tb = W_cat_b[LTAIL:].reshape(8, 128)
    Du, Di, Da, Db = retile(W_user.T, W_item.T, W_cat_a.T, W_cat_b.T,
                            tu, ti, ta, tb)
    gT, lT = gather(Du, Di, Da, Db,
                    user_id.astype(jnp.int32), item_id.astype(jnp.int32),
                    cat_a.astype(jnp.int32), cat_b.astype(jnp.int32))
    return gT.T, lT.T
